# baseline (speedup 1.0000x reference)
import functools

import jax
import jax.numpy as jnp
from jax import lax
from jax.experimental import pallas as pl
from jax.experimental.pallas import tpu as pltpu
from jax.experimental.pallas import tpu_sc as plsc

N = 10000
E = 320000
NC = 2
NS = 16
NW = NC * NS
EPT = E // NW
CH = 80
NCH = EPT // CH
NH = 5
NCH2 = NCH // NH
NPAD = 10112
RPT = NPAD // NS


@functools.lru_cache(maxsize=None)
def _make_spmm(D):
  mesh = plsc.VectorSubcoreMesh(core_axis_name="c", subcore_axis_name="s")

  @functools.partial(
      pl.kernel,
      out_type=jax.ShapeDtypeStruct((NC, NPAD, D), jnp.float32),
      mesh=mesh,
      scratch_types=[
          pltpu.VMEM((2, NCH2, CH), jnp.int32),
          pltpu.VMEM((2, NCH2, CH), jnp.int32),
          pltpu.VMEM((3, CH, D), jnp.float32),
          pltpu.VMEM_SHARED((NPAD, D), jnp.float32),
          [pltpu.SemaphoreType.DMA] * 3,
          [pltpu.SemaphoreType.DMA] * 3,
          pltpu.SemaphoreType.DMA,
      ],
  )
  def spmm(table, src5, dst5, zeros, out,
           src_v, dst_v, rows, acc, gsem, ssem, isem):
    cid = lax.axis_index("c")
    sid = lax.axis_index("s")
    wid = sid * NC + cid

    def g_issue(p, j, b):
      pltpu.async_copy(table.at[src_v.at[p, j]], rows.at[b], gsem[b])

    def g_wait(b):
      pltpu.make_async_copy(table.at[src_v.at[0, 0]], rows.at[b],
                            gsem[b]).wait()

    def s_issue(p, j, b):
      pltpu.async_copy(rows.at[b], acc.at[dst_v.at[p, j]], ssem[b], add=True)

    def s_wait(b):
      pltpu.make_async_copy(rows.at[b], acc.at[dst_v.at[0, 0]],
                            ssem[b]).wait()

    pltpu.sync_copy(src5.at[wid, 0], src_v.at[0])
    pltpu.sync_copy(dst5.at[wid, 0], dst_v.at[0])
    g_issue(0, 0, 0)
    g_issue(0, 1, 1)
    g_issue(0, 2, 2)
    pltpu.sync_copy(zeros.at[pl.ds(sid * RPT, RPT)],
                    acc.at[pl.ds(sid * RPT, RPT)])
    plsc.subcore_barrier()

    for h in range(NH):
      p = h % 2
      if h + 1 < NH:
        pltpu.async_copy(src5.at[wid, h + 1], src_v.at[1 - p], isem)
        pltpu.async_copy(dst5.at[wid, h + 1], dst_v.at[1 - p], isem)
      g_wait(0)
      s_issue(p, 0, 0)

      def body(i, carry):
        for k in range(3):
          j = 1 + 3 * i + k
          b = (1 + k) % 3
          bn = (b + 2) % 3
          g_wait(b)
          s_issue(p, j, b)

          @pl.when(j + 2 < NCH2)
          def _():
            s_wait(bn)
            g_issue(p, j + 2, bn)

        return carry

      lax.fori_loop(0, (NCH2 - 1) // 3, body, 0)
      s_wait(0)
      s_wait(1)
      s_wait(2)
      if h + 1 < NH:
        pltpu.make_async_copy(src5.at[wid, 0], src_v.at[0], isem).wait()
        pltpu.make_async_copy(dst5.at[wid, 0], dst_v.at[0], isem).wait()
        g_issue(1 - p, 0, 0)
        g_issue(1 - p, 1, 1)
        g_issue(1 - p, 2, 2)
    plsc.subcore_barrier()
    pltpu.sync_copy(acc.at[pl.ds(sid * RPT, RPT)],
                    out.at[cid, pl.ds(sid * RPT, RPT)])

  return spmm


BN = 1000


def _self_body(x_ref, s_ref, b_ref, slf_ref):
  slf_ref[...] = (
      jnp.dot(x_ref[...], s_ref[...], preferred_element_type=jnp.float32)
      + b_ref[...])


def _comb_body(agg_ref, slf_ref, w_ref, o_ref):
  a = agg_ref[0] + agg_ref[1]
  o_ref[...] = (
      jnp.dot(a, w_ref[...], preferred_element_type=jnp.float32)
      + slf_ref[...])


def _comb_final_body(agg_ref, slf_ref, w_ref, o_ref):
  a = agg_ref[0] + agg_ref[1]
  x = (jnp.dot(a, w_ref[...], preferred_element_type=jnp.float32)
       + slf_ref[...])
  m = jnp.max(x, axis=1, keepdims=True)
  e = jnp.exp(x - m)
  lse = jnp.log(jnp.sum(e, axis=1, keepdims=True)) + m
  o_ref[...] = x - lse


@functools.lru_cache(maxsize=None)
def _make_self(DI, DO):
  return pl.pallas_call(
      _self_body,
      grid=(N // BN,),
      in_specs=[
          pl.BlockSpec((BN, DI), lambda i: (i, 0)),
          pl.BlockSpec((DI, DO), lambda i: (0, 0)),
          pl.BlockSpec((1, DO), lambda i: (0, 0)),
      ],
      out_specs=pl.BlockSpec((BN, DO), lambda i: (i, 0)),
      out_shape=jax.ShapeDtypeStruct((N, DO), jnp.float32),
  )


@functools.lru_cache(maxsize=None)
def _make_comb(DO, final):
  return pl.pallas_call(
      _comb_final_body if final else _comb_body,
      grid=(N // BN,),
      in_specs=[
          pl.BlockSpec((NC, BN, 128), lambda i: (0, i, 0)),
          pl.BlockSpec((BN, DO), lambda i: (i, 0)),
          pl.BlockSpec((128, DO), lambda i: (0, 0)),
      ],
      out_specs=pl.BlockSpec((BN, DO), lambda i: (i, 0)),
      out_shape=jax.ShapeDtypeStruct((N, DO), jnp.float32),
  )


def kernel(fea, adj, W0, S0, b0, W1, S1, b1, W2, S2, b2, W3, S3, b3):
  src3 = adj[0].reshape(NW, NH, NCH2, CH)
  dst3 = adj[1].reshape(NW, NH, NCH2, CH)
  z128 = jnp.zeros((NPAD, 128), jnp.float32)
  spmm = _make_spmm(128)

  x = fea
  for W, S, b, DO in ((W0, S0, b0, 128), (W1, S1, b1, 128),
                      (W2, S2, b2, 128)):
    slf = _make_self(128, DO)(x, S, b.reshape(1, -1))
    agg = spmm(x, src3, dst3, z128)
    x = _make_comb(DO, False)(agg, slf, W)
  slf = _make_self(128, 64)(x, S3, b3.reshape(1, -1))
  agg = spmm(x, src3, dst3, z128)
  return _make_comb(64, True)(agg, slf, W3)


# speedup vs baseline: 1.0343x; 1.0343x over previous
import functools

import jax
import jax.numpy as jnp
from jax import lax
from jax.experimental import pallas as pl
from jax.experimental.pallas import tpu as pltpu
from jax.experimental.pallas import tpu_sc as plsc

N = 10000
E = 320000
NC = 2
NS = 16
NW = NC * NS
EPT = E // NW
CH = 80
NCH = EPT // CH
NH = 5
NCH2 = NCH // NH
NPAD = 10112
RPT = NPAD // NS


@functools.lru_cache(maxsize=None)
def _make_spmm(D):
  mesh = plsc.VectorSubcoreMesh(core_axis_name="c", subcore_axis_name="s")

  @functools.partial(
      pl.kernel,
      out_type=jax.ShapeDtypeStruct((NC, NPAD, D), jnp.float32),
      mesh=mesh,
      scratch_types=[
          pltpu.VMEM((2, NCH2, CH), jnp.int32),
          pltpu.VMEM((2, NCH2, CH), jnp.int32),
          pltpu.VMEM((3, CH, D), jnp.float32),
          pltpu.VMEM_SHARED((NPAD, D), jnp.float32),
          [pltpu.SemaphoreType.DMA] * 3,
          [pltpu.SemaphoreType.DMA] * 3,
          pltpu.SemaphoreType.DMA,
      ],
  )
  def spmm(table, src5, dst5, zeros, out,
           src_v, dst_v, rows, acc, gsem, ssem, isem):
    cid = lax.axis_index("c")
    sid = lax.axis_index("s")
    wid = sid * NC + cid

    def g_issue(p, j, b):
      pltpu.async_copy(table.at[src_v.at[p, j]], rows.at[b], gsem[b])

    def g_wait(b):
      pltpu.make_async_copy(table.at[src_v.at[0, 0]], rows.at[b],
                            gsem[b]).wait()

    def s_issue(p, j, b):
      pltpu.async_copy(rows.at[b], acc.at[dst_v.at[p, j]], ssem[b], add=True)

    def s_wait(b):
      pltpu.make_async_copy(rows.at[b], acc.at[dst_v.at[0, 0]],
                            ssem[b]).wait()

    pltpu.sync_copy(src5.at[wid, 0], src_v.at[0])
    pltpu.sync_copy(dst5.at[wid, 0], dst_v.at[0])
    g_issue(0, 0, 0)
    g_issue(0, 1, 1)
    pltpu.sync_copy(zeros.at[pl.ds(sid * RPT, RPT)],
                    acc.at[pl.ds(sid * RPT, RPT)])
    plsc.subcore_barrier()

    for h in range(NH):
      p = h % 2
      if h + 1 < NH:
        pltpu.async_copy(src5.at[wid, h + 1], src_v.at[1 - p], isem)
        pltpu.async_copy(dst5.at[wid, h + 1], dst_v.at[1 - p], isem)
      g_wait(0)
      s_issue(p, 0, 0)
      g_issue(p, 2, 2)

      def body(i, carry):
        for k in range(3):
          j = 1 + 3 * i + k
          b = (1 + k) % 3
          bn = (b + 2) % 3
          g_wait(b)
          s_issue(p, j, b)

          @pl.when(j + 2 < NCH2)
          def _():
            s_wait(bn)
            g_issue(p, j + 2, bn)

        return carry

      lax.fori_loop(0, (NCH2 - 1) // 3, body, 0)
      s_wait(0)
      s_wait(1)
      s_wait(2)
      if h + 1 < NH:
        pltpu.make_async_copy(src5.at[wid, 0], src_v.at[0], isem).wait()
        pltpu.make_async_copy(dst5.at[wid, 0], dst_v.at[0], isem).wait()
        g_issue(1 - p, 0, 0)
        g_issue(1 - p, 1, 1)
    plsc.subcore_barrier()
    pltpu.sync_copy(acc.at[pl.ds(sid * RPT, RPT)],
                    out.at[cid, pl.ds(sid * RPT, RPT)])

  return spmm


BN = 1000


def _self_body(x_ref, s_ref, b_ref, slf_ref):
  slf_ref[...] = (
      jnp.dot(x_ref[...], s_ref[...], preferred_element_type=jnp.float32)
      + b_ref[...])


def _comb_body(agg_ref, slf_ref, w_ref, o_ref):
  a = agg_ref[0] + agg_ref[1]
  o_ref[...] = (
      jnp.dot(a, w_ref[...], preferred_element_type=jnp.float32)
      + slf_ref[...])


def _comb_final_body(agg_ref, slf_ref, w_ref, o_ref):
  a = agg_ref[0] + agg_ref[1]
  x = (jnp.dot(a, w_ref[...], preferred_element_type=jnp.float32)
       + slf_ref[...])
  m = jnp.max(x, axis=1, keepdims=True)
  e = jnp.exp(x - m)
  lse = jnp.log(jnp.sum(e, axis=1, keepdims=True)) + m
  o_ref[...] = x - lse


@functools.lru_cache(maxsize=None)
def _make_self(DI, DO):
  return pl.pallas_call(
      _self_body,
      grid=(N // BN,),
      in_specs=[
          pl.BlockSpec((BN, DI), lambda i: (i, 0)),
          pl.BlockSpec((DI, DO), lambda i: (0, 0)),
          pl.BlockSpec((1, DO), lambda i: (0, 0)),
      ],
      out_specs=pl.BlockSpec((BN, DO), lambda i: (i, 0)),
      out_shape=jax.ShapeDtypeStruct((N, DO), jnp.float32),
  )


@functools.lru_cache(maxsize=None)
def _make_comb(DO, final):
  return pl.pallas_call(
      _comb_final_body if final else _comb_body,
      grid=(N // BN,),
      in_specs=[
          pl.BlockSpec((NC, BN, 128), lambda i: (0, i, 0)),
          pl.BlockSpec((BN, DO), lambda i: (i, 0)),
          pl.BlockSpec((128, DO), lambda i: (0, 0)),
      ],
      out_specs=pl.BlockSpec((BN, DO), lambda i: (i, 0)),
      out_shape=jax.ShapeDtypeStruct((N, DO), jnp.float32),
  )


def kernel(fea, adj, W0, S0, b0, W1, S1, b1, W2, S2, b2, W3, S3, b3):
  src3 = adj[0].reshape(NW, NH, NCH2, CH)
  dst3 = adj[1].reshape(NW, NH, NCH2, CH)
  z128 = jnp.zeros((NPAD, 128), jnp.float32)
  spmm = _make_spmm(128)

  x = fea
  for W, S, b, DO in ((W0, S0, b0, 128), (W1, S1, b1, 128),
                      (W2, S2, b2, 128)):
    slf = _make_self(128, DO)(x, S, b.reshape(1, -1))
    agg = spmm(x, src3, dst3, z128)
    x = _make_comb(DO, False)(agg, slf, W)
  slf = _make_self(128, 64)(x, S3, b3.reshape(1, -1))
  agg = spmm(x, src3, dst3, z128)
  return _make_comb(64, True)(agg, slf, W3)
